# Initial kernel scaffold; baseline (speedup 1.0000x reference)
import jax
import jax.numpy as jnp
from jax.experimental import pallas as pl


def kernel(x, edge_index, edge_weights):
    raise NotImplementedError("write your pallas kernel here")


# speedup vs baseline: 2.2471x; 2.2471x over previous
import functools

import jax
import jax.numpy as jnp
from jax import lax
from jax.experimental import pallas as pl
from jax.experimental.pallas import tpu as pltpu
from jax.experimental.pallas import tpu_sc as plsc

N_NODES = 10000
D = 128
E_EDGES = 160000
NUM_ITERS = 75
K_NODES = 5
ADJ_THRESH = 0.5

NC, NS = 2, 16
NTILES = NC * NS
WIN = 240
CH = 120
RB = 64
EPAD = E_EDGES + 2 * CH


def _shard_bounds():
    bounds = []
    per_sc = E_EDGES // NC
    for c in range(NC):
        e0, e1 = c * per_sc, (c + 1) * per_sc
        nw = -(-(e1 - e0) // WIN)
        base, rem = divmod(nw, NS)
        pos = e0
        for t in range(NS):
            w = base + (1 if t < rem else 0)
            nxt = min(pos + w * WIN, e1)
            bounds.append((pos, nxt))
            pos = nxt
    return bounds


SHARDS = _shard_bounds()


def _sc_scatter_body(h_hbm, srcp_hbm, dstp_hbm, wnp_hbm, meta_hbm,
                     out_hbm, stgrows_hbm, stgids_hbm,
                     sidx, rows, accbuf, zrow, tmprow, idrow,
                     dsm, wsm, msm, gsem, rsem, zsem):
    c = lax.axis_index("c")
    s = lax.axis_index("s")
    wid = c * NS + s

    def sload(ref, i):
        return ref[pl.ds(i, 16)][0]

    pltpu.sync_copy(meta_hbm, msm.at[pl.ds(0, 96)])
    e0 = sload(msm, wid)
    e1 = sload(msm, 32 + wid)
    cov_hi = sload(msm, 64 + wid)

    zeros16 = jnp.zeros((16,), jnp.float32)
    for j in range(8):
        zrow.at[0][pl.ds(16 * j, 16)] = zeros16

    nedge = e1 - e0
    nch = lax.div(nedge + (CH - 1), CH)

    def stage_write(slot_off, node, acc):
        for j in range(8):
            tmprow.at[0][pl.ds(16 * j, 16)] = acc[j]
        pltpu.sync_copy(tmprow, stgrows_hbm.at[pl.ds(2 * wid + slot_off, 1)])
        idrow.at[0][...] = jnp.full((16,), node, jnp.int32)
        pltpu.sync_copy(idrow, stgids_hbm.at[pl.ds(2 * wid + slot_off, 1)])

    def chunk_body(ci, carry):
        cur, run_idx, slot, nfired, gcount = carry[:5]
        acc = carry[5:]
        e = pl.multiple_of(e0 + ci * CH, 8)
        pltpu.sync_copy(srcp_hbm.at[pl.ds(e, CH)], sidx)
        pltpu.sync_copy(dstp_hbm.at[pl.ds(e, CH)], dsm.at[pl.ds(0, CH)])
        pltpu.sync_copy(wnp_hbm.at[pl.ds(e, CH)], wsm.at[pl.ds(0, CH)])
        pltpu.async_copy(h_hbm.at[sidx], rows, gsem).wait()
        m = jnp.minimum(CH, nedge - ci * CH)

        def edge_body(i, ecarry):
            cur, run_idx, slot, nfired, gcount = ecarry[:5]
            acc = list(ecarry[5:])
            n = sload(dsm, i)
            wv = sload(wsm, i)
            switch = jnp.logical_and(n != cur, True)

            @pl.when(jnp.logical_and(switch, cur >= 0))
            def _():
                @pl.when(run_idx == 0)
                def _():
                    stage_write(0, cur, acc)

                @pl.when(run_idx > 0)
                def _():
                    for j in range(8):
                        accbuf.at[slot][pl.ds(16 * j, 16)] = acc[j]
                    pltpu.async_copy(accbuf.at[pl.ds(slot, 1)],
                                     out_hbm.at[pl.ds(cur, 1)], rsem)

            gap_lo = jnp.where(cur >= 0, cur + 1, jnp.where(wid == 0, 0, n))

            @pl.when(jnp.logical_and(switch, n > gap_lo))
            def _():
                def gfill(g, _):
                    pltpu.async_copy(zrow, out_hbm.at[pl.ds(g, 1)], zsem)
                    return 0
                lax.fori_loop(gap_lo, n, gfill, 0)

            emit = jnp.logical_and(switch, jnp.logical_and(cur >= 0, run_idx > 0))
            new_slot = jnp.where(emit, lax.rem(slot + 1, RB), slot)
            new_nfired = jnp.where(emit, nfired + 1, nfired)

            @pl.when(jnp.logical_and(emit, new_slot == 0))
            def _():
                pltpu.make_async_copy(out_hbm.at[pl.ds(0, RB)], accbuf, rsem).wait()

            new_gcount = gcount + jnp.where(
                jnp.logical_and(switch, n > gap_lo), n - gap_lo, 0)
            new_run_idx = jnp.where(jnp.logical_and(switch, cur >= 0),
                                    run_idx + 1, run_idx)
            new_cur = jnp.where(switch, n, cur)

            new_acc = []
            for j in range(8):
                base = jnp.where(switch, zeros16, acc[j])
                new_acc.append(base + rows.at[i][pl.ds(16 * j, 16)] * wv)
            return (new_cur, new_run_idx, new_slot, new_nfired, new_gcount,
                    *new_acc)

        return lax.fori_loop(0, m, edge_body,
                             (cur, run_idx, slot, nfired, gcount, *acc))

    init = (jnp.int32(-1), jnp.int32(0), jnp.int32(0), jnp.int32(0),
            jnp.int32(0), *[zeros16 for _ in range(8)])
    fin = lax.fori_loop(0, nch, chunk_body, init)
    cur, run_idx, slot, nfired, gcount = fin[:5]
    acc = fin[5:]

    @pl.when(run_idx == 0)
    def _():
        stage_write(0, cur, acc)
        idrow.at[0][...] = jnp.full((16,), -1, jnp.int32)
        pltpu.sync_copy(idrow, stgids_hbm.at[pl.ds(2 * wid + 1, 1)])

    @pl.when(run_idx > 0)
    def _():
        stage_write(1, cur, acc)

    @pl.when(cov_hi > cur + 1)
    def _():
        def gfill(g, _):
            pltpu.async_copy(zrow, out_hbm.at[pl.ds(g, 1)], zsem)
            return 0
        lax.fori_loop(cur + 1, cov_hi, gfill, 0)
    gcount = gcount + jnp.maximum(cov_hi - (cur + 1), 0)

    rem = lax.rem(nfired, RB)

    def drain(i, _):
        pltpu.make_async_copy(out_hbm.at[pl.ds(0, 1)],
                              accbuf.at[pl.ds(0, 1)], rsem).wait()
        return 0
    lax.fori_loop(0, rem, drain, 0)

    def draing(i, _):
        pltpu.make_async_copy(out_hbm.at[pl.ds(0, 1)],
                              zrow, zsem).wait()
        return 0
    lax.fori_loop(0, gcount, draing, 0)


_sc_scatter = pl.kernel(
    _sc_scatter_body,
    out_type=(
        jax.ShapeDtypeStruct((N_NODES, D), jnp.float32),
        jax.ShapeDtypeStruct((2 * NTILES, D), jnp.float32),
        jax.ShapeDtypeStruct((2 * NTILES, 16), jnp.int32),
    ),
    mesh=plsc.VectorSubcoreMesh(core_axis_name="c", subcore_axis_name="s",
                                num_cores=NC, num_subcores=NS),
    scratch_types=[
        pltpu.VMEM((CH,), jnp.int32),
        pltpu.VMEM((CH, D), jnp.float32),
        pltpu.VMEM((RB, D), jnp.float32),
        pltpu.VMEM((1, D), jnp.float32),
        pltpu.VMEM((1, D), jnp.float32),
        pltpu.VMEM((1, 16), jnp.int32),
        pltpu.VMEM((CH + 16,), jnp.int32),
        pltpu.VMEM((CH + 16,), jnp.float32),
        pltpu.VMEM((112,), jnp.int32),
        pltpu.SemaphoreType.DMA,
        pltpu.SemaphoreType.DMA,
        pltpu.SemaphoreType.DMA,
    ],
)


def _merge_body(hraw_ref, stgrows_ref, stgids_ref, out_ref):
    zero = jnp.zeros((1, D), jnp.float32)
    out_ref[...] = hraw_ref[...]

    def body(k, carry):
        cur, svec = carry
        nid = stgids_ref[k, 0]
        p = stgrows_ref[pl.ds(k, 1), :]
        valid = nid >= 0
        same = jnp.logical_and(valid, nid == cur)
        switch = jnp.logical_and(valid, nid != cur)

        @pl.when(jnp.logical_and(switch, cur >= 0))
        def _():
            out_ref[pl.ds(cur, 1), :] = svec

        new_svec = jnp.where(same, svec + p, jnp.where(valid, zero + p, svec))
        new_cur = jnp.where(switch, nid, cur)
        return (new_cur, new_svec)

    cur, svec = lax.fori_loop(0, 2 * NTILES, body, (jnp.int32(-1), zero))

    @pl.when(cur >= 0)
    def _():
        out_ref[pl.ds(cur, 1), :] = svec


_merge = pl.pallas_call(
    _merge_body,
    out_shape=jax.ShapeDtypeStruct((N_NODES, D), jnp.float32),
    in_specs=[
        pl.BlockSpec((N_NODES, D), lambda: (0, 0)),
        pl.BlockSpec((2 * NTILES, D), lambda: (0, 0)),
        pl.BlockSpec(memory_space=pltpu.SMEM),
    ],
    out_specs=pl.BlockSpec((N_NODES, D), lambda: (0, 0)),
)


BR = 400


def _norm_body(h_ref, o_ref):
    x = h_ref[...]
    acc = x[:, 0:8] * x[:, 0:8]
    for i in range(1, 16):
        ci = x[:, 8 * i:8 * i + 8]
        acc = acc + ci * ci
    t = acc[:, 0:4] + acc[:, 4:8]
    t = t[:, 0:2] + t[:, 2:4]
    ss = t[:, 0:1] + t[:, 1:2]
    nrm = jnp.sqrt(ss)
    den = jnp.maximum(nrm, 1e-12)
    o_ref[...] = x / den


_normalize = pl.pallas_call(
    _norm_body,
    out_shape=jax.ShapeDtypeStruct((N_NODES, D), jnp.float32),
    grid=(N_NODES // BR,),
    in_specs=[pl.BlockSpec((BR, D), lambda i: (i, 0))],
    out_specs=pl.BlockSpec((BR, D), lambda i: (i, 0)),
)


def _scores_body(h_ref, o_ref):
    o_ref[...] = jnp.max(h_ref[...], axis=1, keepdims=True)


_scores_k = pl.pallas_call(
    _scores_body,
    out_shape=jax.ShapeDtypeStruct((N_NODES, 1), jnp.float32),
    grid=(N_NODES // BR,),
    in_specs=[pl.BlockSpec((BR, D), lambda i: (i, 0))],
    out_specs=pl.BlockSpec((BR, 1), lambda i: (i, 0)),
)


def _extract_body(pf_ref, hn_ref, x_ref, sc_ref,
                  rf_ref, masks_ref, nf_ref, si_ref, ns_ref):
    s = sc_ref[...]
    b = lax.bitcast_convert_type(s, jnp.int32)
    m = b ^ ((b >> 31) & jnp.int32(0x7FFFFFFF))
    riota = lax.broadcasted_iota(jnp.int32, (N_NODES, 1), 0)
    neg_inf_i = jnp.int32(-0x80000000)

    idxs = []
    for k in range(K_NODES):
        mv = jnp.max(m)
        cand = jnp.where(m == mv, riota, jnp.int32(N_NODES))
        mi = jnp.min(cand)
        idxs.append(mi)
        si_ref[k] = mi
        m = jnp.where(riota == mi, neg_inf_i, m)
    for k in range(K_NODES, 8):
        si_ref[k] = jnp.int32(0)

    raw = jnp.concatenate(
        [pf_ref[pl.ds(idxs[k], 1), :] for k in range(K_NODES)], axis=0)
    rf_ref[...] = raw

    rss = jnp.sum(raw * raw, axis=1, keepdims=True)
    rn = raw / jnp.maximum(jnp.sqrt(rss), 1e-12)

    hn = hn_ref[...]
    sim = lax.dot_general(rn, hn, (((1,), (1,)), ((), ())),
                          preferred_element_type=jnp.float32,
                          precision=lax.Precision.HIGHEST)
    mx = jnp.max(sim, axis=1, keepdims=True)
    ex = jnp.exp(sim - mx)
    sm = ex / jnp.sum(ex, axis=1, keepdims=True)
    sm = sm / jnp.maximum(jnp.max(sm, axis=1, keepdims=True), 1e-12)
    masks_ref[...] = sm

    nf = lax.dot_general(sm, x_ref[...], (((1,), (0,)), ((), ())),
                         preferred_element_type=jnp.float32,
                         precision=lax.Precision.HIGHEST)
    nf_ref[...] = nf
    for k in range(K_NODES):
        ns_ref[k] = jnp.max(sm[k])
    for k in range(K_NODES, 8):
        ns_ref[k] = jnp.float32(0)


_extract = pl.pallas_call(
    _extract_body,
    out_shape=(
        jax.ShapeDtypeStruct((K_NODES, D), jnp.float32),
        jax.ShapeDtypeStruct((K_NODES, N_NODES), jnp.float32),
        jax.ShapeDtypeStruct((K_NODES, D), jnp.float32),
        jax.ShapeDtypeStruct((8,), jnp.int32),
        jax.ShapeDtypeStruct((8,), jnp.float32),
    ),
    out_specs=(
        pl.BlockSpec((K_NODES, D), lambda: (0, 0)),
        pl.BlockSpec((K_NODES, N_NODES), lambda: (0, 0)),
        pl.BlockSpec((K_NODES, D), lambda: (0, 0)),
        pl.BlockSpec(memory_space=pltpu.SMEM),
        pl.BlockSpec(memory_space=pltpu.SMEM),
    ),
)


def kernel(x, edge_index, edge_weights):
    src = edge_index[0]
    dst = edge_index[1]

    w = jnp.where(edge_weights > ADJ_THRESH, edge_weights, 0.0)
    deg = jax.ops.segment_sum(w, dst, num_segments=N_NODES)
    wn = w / jnp.clip(deg[dst], 1e-12)

    perm = jnp.argsort(dst, stable=True)
    srcp = src[perm].astype(jnp.int32)
    dstp = dst[perm].astype(jnp.int32)
    wnp = wn[perm]

    pad = EPAD - E_EDGES
    srcp = jnp.concatenate([srcp, jnp.zeros((pad,), jnp.int32)])
    dstp_p = jnp.concatenate([dstp, jnp.zeros((pad,), jnp.int32)])
    wnp = jnp.concatenate([wnp, jnp.zeros((pad,), jnp.float32)])

    e0s = jnp.array([sh[0] for sh in SHARDS], jnp.int32)
    e1s = jnp.array([sh[1] for sh in SHARDS], jnp.int32)
    nxt = jnp.concatenate([
        dstp[jnp.array([sh[1] for sh in SHARDS[:-1]], jnp.int32)],
        jnp.array([N_NODES], jnp.int32)])
    meta = jnp.concatenate([e0s, e1s, nxt])

    h0 = jax.random.normal(jax.random.key(42), x.shape, dtype=jnp.float32)

    def body(i, h):
        hraw, stgrows, stgids = _sc_scatter(h, srcp, dstp_p, wnp, meta)
        hm = _merge(hraw, stgrows, stgids)
        return _normalize(hm)

    pf = lax.fori_loop(0, NUM_ITERS, body, h0)

    scores = _scores_k(pf)
    hn = _normalize(pf)
    rf, masks, nf, si8, ns8 = _extract(pf, hn, x, scores)
    return (nf, ns8[:K_NODES], masks, rf, si8[:K_NODES])


# speedup vs baseline: 2.8480x; 1.2674x over previous
import functools

import jax
import jax.numpy as jnp
from jax import lax
from jax.experimental import pallas as pl
from jax.experimental.pallas import tpu as pltpu
from jax.experimental.pallas import tpu_sc as plsc

N_NODES = 10000
D = 128
E_EDGES = 160000
NUM_ITERS = 75
K_NODES = 5
ADJ_THRESH = 0.5

NC, NS = 2, 16
NTILES = NC * NS
WIN = 240
CH = 120
RB = 64
EPAD = E_EDGES + 512
SHMAX = 5040


def _shard_bounds():
    bounds = []
    per_sc = E_EDGES // NC
    for c in range(NC):
        e0, e1 = c * per_sc, (c + 1) * per_sc
        nw = -(-(e1 - e0) // WIN)
        base, rem = divmod(nw, NS)
        pos = e0
        for t in range(NS):
            w = base + (1 if t < rem else 0)
            nxt = min(pos + w * WIN, e1)
            bounds.append((pos, nxt))
            pos = nxt
    return bounds


SHARDS = _shard_bounds()


def _sc_scatter_body(h_hbm, srcp_hbm, dstp_hbm, wnp_hbm, meta_hbm,
                     out_hbm, stgrows_hbm, stgids_hbm,
                     srcb, dstb, wnb, rows, accbuf, zrow, tmprow, idrow,
                     msm, gsem, rsem, zsem):
    c = lax.axis_index("c")
    s = lax.axis_index("s")
    wid = c * NS + s

    def sload(ref, i):
        return ref[pl.ds(i, 16)][0]

    pltpu.sync_copy(meta_hbm, msm.at[pl.ds(0, 96)])
    e0 = sload(msm, wid)
    e1 = sload(msm, 32 + wid)
    cov_hi = sload(msm, 64 + wid)

    zeros16 = jnp.zeros((16,), jnp.float32)
    for j in range(8):
        zrow.at[0][pl.ds(16 * j, 16)] = zeros16

    nedge = e1 - e0
    nch = lax.div(nedge + (CH - 1), CH)

    e0a = pl.multiple_of(e0, 8)
    pltpu.sync_copy(srcp_hbm.at[pl.ds(e0a, SHMAX)], srcb.at[pl.ds(0, SHMAX)])
    pltpu.sync_copy(dstp_hbm.at[pl.ds(e0a, SHMAX)], dstb.at[pl.ds(0, SHMAX)])
    pltpu.sync_copy(wnp_hbm.at[pl.ds(e0a, SHMAX)], wnb.at[pl.ds(0, SHMAX)])

    def issue_gather(k):
        off = pl.multiple_of(k * CH, 8)
        return pltpu.async_copy(h_hbm.at[srcb.at[pl.ds(off, CH)]],
                                rows.at[lax.rem(k, 2)], gsem.at[lax.rem(k, 2)])

    issue_gather(0)

    def stage_write(slot_off, node, acc):
        for j in range(8):
            tmprow.at[0][pl.ds(16 * j, 16)] = acc[j]
        pltpu.sync_copy(tmprow, stgrows_hbm.at[pl.ds(2 * wid + slot_off, 1)])
        idrow.at[0][...] = jnp.full((16,), node, jnp.int32)
        pltpu.sync_copy(idrow, stgids_hbm.at[pl.ds(2 * wid + slot_off, 1)])

    def chunk_body(ci, carry):
        cur, run_idx, slot, nfired, gcount = carry[:5]
        acc = carry[5:]
        p = lax.rem(ci, 2)
        off = pl.multiple_of(ci * CH, 8)
        pltpu.make_async_copy(h_hbm.at[srcb.at[pl.ds(off, CH)]],
                              rows.at[p], gsem.at[p]).wait()

        @pl.when(ci + 1 < nch)
        def _():
            issue_gather(ci + 1)
        m = jnp.minimum(CH, nedge - ci * CH)

        def edge_body(i, ecarry):
            cur, run_idx, slot, nfired, gcount = ecarry[:5]
            acc = list(ecarry[5:])
            n = sload(dstb, off + i)
            wv = sload(wnb, off + i)
            switch = jnp.logical_and(n != cur, True)

            @pl.when(jnp.logical_and(switch, cur >= 0))
            def _():
                @pl.when(run_idx == 0)
                def _():
                    stage_write(0, cur, acc)

                @pl.when(run_idx > 0)
                def _():
                    for j in range(8):
                        accbuf.at[slot][pl.ds(16 * j, 16)] = acc[j]
                    pltpu.async_copy(accbuf.at[pl.ds(slot, 1)],
                                     out_hbm.at[pl.ds(cur, 1)], rsem)

            gap_lo = jnp.where(cur >= 0, cur + 1, jnp.where(wid == 0, 0, n))

            @pl.when(jnp.logical_and(switch, n > gap_lo))
            def _():
                def gfill(g, _):
                    pltpu.async_copy(zrow, out_hbm.at[pl.ds(g, 1)], zsem)
                    return 0
                lax.fori_loop(gap_lo, n, gfill, 0)

            emit = jnp.logical_and(switch, jnp.logical_and(cur >= 0, run_idx > 0))
            new_slot = jnp.where(emit, lax.rem(slot + 1, RB), slot)
            new_nfired = jnp.where(emit, nfired + 1, nfired)

            @pl.when(jnp.logical_and(emit, new_slot == 0))
            def _():
                pltpu.make_async_copy(out_hbm.at[pl.ds(0, RB)], accbuf, rsem).wait()

            new_gcount = gcount + jnp.where(
                jnp.logical_and(switch, n > gap_lo), n - gap_lo, 0)
            new_run_idx = jnp.where(jnp.logical_and(switch, cur >= 0),
                                    run_idx + 1, run_idx)
            new_cur = jnp.where(switch, n, cur)

            new_acc = []
            for j in range(8):
                base = jnp.where(switch, zeros16, acc[j])
                new_acc.append(base + rows.at[p].at[i][pl.ds(16 * j, 16)] * wv)
            return (new_cur, new_run_idx, new_slot, new_nfired, new_gcount,
                    *new_acc)

        return lax.fori_loop(0, m, edge_body,
                             (cur, run_idx, slot, nfired, gcount, *acc))

    init = (jnp.int32(-1), jnp.int32(0), jnp.int32(0), jnp.int32(0),
            jnp.int32(0), *[zeros16 for _ in range(8)])
    fin = lax.fori_loop(0, nch, chunk_body, init)
    cur, run_idx, slot, nfired, gcount = fin[:5]
    acc = fin[5:]

    @pl.when(run_idx == 0)
    def _():
        stage_write(0, cur, acc)
        idrow.at[0][...] = jnp.full((16,), -1, jnp.int32)
        pltpu.sync_copy(idrow, stgids_hbm.at[pl.ds(2 * wid + 1, 1)])

    @pl.when(run_idx > 0)
    def _():
        stage_write(1, cur, acc)

    @pl.when(cov_hi > cur + 1)
    def _():
        def gfill(g, _):
            pltpu.async_copy(zrow, out_hbm.at[pl.ds(g, 1)], zsem)
            return 0
        lax.fori_loop(cur + 1, cov_hi, gfill, 0)
    gcount = gcount + jnp.maximum(cov_hi - (cur + 1), 0)

    rem = lax.rem(nfired, RB)

    def drain(i, _):
        pltpu.make_async_copy(out_hbm.at[pl.ds(0, 1)],
                              accbuf.at[pl.ds(0, 1)], rsem).wait()
        return 0
    lax.fori_loop(0, rem, drain, 0)

    def draing(i, _):
        pltpu.make_async_copy(out_hbm.at[pl.ds(0, 1)],
                              zrow, zsem).wait()
        return 0
    lax.fori_loop(0, gcount, draing, 0)


_sc_scatter = pl.kernel(
    _sc_scatter_body,
    out_type=(
        jax.ShapeDtypeStruct((N_NODES, D), jnp.float32),
        jax.ShapeDtypeStruct((2 * NTILES, D), jnp.float32),
        jax.ShapeDtypeStruct((2 * NTILES, 16), jnp.int32),
    ),
    mesh=plsc.VectorSubcoreMesh(core_axis_name="c", subcore_axis_name="s",
                                num_cores=NC, num_subcores=NS),
    scratch_types=[
        pltpu.VMEM((SHMAX + 16,), jnp.int32),
        pltpu.VMEM((SHMAX + 16,), jnp.int32),
        pltpu.VMEM((SHMAX + 16,), jnp.float32),
        pltpu.VMEM((2, CH, D), jnp.float32),
        pltpu.VMEM((RB, D), jnp.float32),
        pltpu.VMEM((1, D), jnp.float32),
        pltpu.VMEM((1, D), jnp.float32),
        pltpu.VMEM((1, 16), jnp.int32),
        pltpu.VMEM((112,), jnp.int32),
        pltpu.SemaphoreType.DMA((2,)),
        pltpu.SemaphoreType.DMA,
        pltpu.SemaphoreType.DMA,
    ],
)


def _merge_body(hraw_ref, stgrows_ref, stgids_ref, out_ref):
    zero = jnp.zeros((1, D), jnp.float32)
    out_ref[...] = hraw_ref[...]

    def body(k, carry):
        cur, svec = carry
        nid = stgids_ref[k, 0]
        p = stgrows_ref[pl.ds(k, 1), :]
        valid = nid >= 0
        same = jnp.logical_and(valid, nid == cur)
        switch = jnp.logical_and(valid, nid != cur)

        @pl.when(jnp.logical_and(switch, cur >= 0))
        def _():
            out_ref[pl.ds(cur, 1), :] = svec

        new_svec = jnp.where(same, svec + p, jnp.where(valid, zero + p, svec))
        new_cur = jnp.where(switch, nid, cur)
        return (new_cur, new_svec)

    cur, svec = lax.fori_loop(0, 2 * NTILES, body, (jnp.int32(-1), zero))

    @pl.when(cur >= 0)
    def _():
        out_ref[pl.ds(cur, 1), :] = svec


_merge = pl.pallas_call(
    _merge_body,
    out_shape=jax.ShapeDtypeStruct((N_NODES, D), jnp.float32),
    in_specs=[
        pl.BlockSpec((N_NODES, D), lambda: (0, 0)),
        pl.BlockSpec((2 * NTILES, D), lambda: (0, 0)),
        pl.BlockSpec(memory_space=pltpu.SMEM),
    ],
    out_specs=pl.BlockSpec((N_NODES, D), lambda: (0, 0)),
)


BR = 400


def _norm_body(h_ref, o_ref):
    x = h_ref[...]
    acc = x[:, 0:8] * x[:, 0:8]
    for i in range(1, 16):
        ci = x[:, 8 * i:8 * i + 8]
        acc = acc + ci * ci
    t = acc[:, 0:4] + acc[:, 4:8]
    t = t[:, 0:2] + t[:, 2:4]
    ss = t[:, 0:1] + t[:, 1:2]
    nrm = jnp.sqrt(ss)
    den = jnp.maximum(nrm, 1e-12)
    o_ref[...] = x / den


_normalize = pl.pallas_call(
    _norm_body,
    out_shape=jax.ShapeDtypeStruct((N_NODES, D), jnp.float32),
    grid=(N_NODES // BR,),
    in_specs=[pl.BlockSpec((BR, D), lambda i: (i, 0))],
    out_specs=pl.BlockSpec((BR, D), lambda i: (i, 0)),
)


def _scores_body(h_ref, o_ref):
    o_ref[...] = jnp.max(h_ref[...], axis=1, keepdims=True)


_scores_k = pl.pallas_call(
    _scores_body,
    out_shape=jax.ShapeDtypeStruct((N_NODES, 1), jnp.float32),
    grid=(N_NODES // BR,),
    in_specs=[pl.BlockSpec((BR, D), lambda i: (i, 0))],
    out_specs=pl.BlockSpec((BR, 1), lambda i: (i, 0)),
)


def _extract_body(pf_ref, hn_ref, x_ref, sc_ref,
                  rf_ref, masks_ref, nf_ref, si_ref, ns_ref):
    s = sc_ref[...]
    b = lax.bitcast_convert_type(s, jnp.int32)
    m = b ^ ((b >> 31) & jnp.int32(0x7FFFFFFF))
    riota = lax.broadcasted_iota(jnp.int32, (N_NODES, 1), 0)
    neg_inf_i = jnp.int32(-0x80000000)

    idxs = []
    for k in range(K_NODES):
        mv = jnp.max(m)
        cand = jnp.where(m == mv, riota, jnp.int32(N_NODES))
        mi = jnp.min(cand)
        idxs.append(mi)
        si_ref[k] = mi
        m = jnp.where(riota == mi, neg_inf_i, m)
    for k in range(K_NODES, 8):
        si_ref[k] = jnp.int32(0)

    raw = jnp.concatenate(
        [pf_ref[pl.ds(idxs[k], 1), :] for k in range(K_NODES)], axis=0)
    rf_ref[...] = raw

    rss = jnp.sum(raw * raw, axis=1, keepdims=True)
    rn = raw / jnp.maximum(jnp.sqrt(rss), 1e-12)

    hn = hn_ref[...]
    sim = lax.dot_general(rn, hn, (((1,), (1,)), ((), ())),
                          preferred_element_type=jnp.float32,
                          precision=lax.Precision.HIGHEST)
    mx = jnp.max(sim, axis=1, keepdims=True)
    ex = jnp.exp(sim - mx)
    sm = ex / jnp.sum(ex, axis=1, keepdims=True)
    sm = sm / jnp.maximum(jnp.max(sm, axis=1, keepdims=True), 1e-12)
    masks_ref[...] = sm

    nf = lax.dot_general(sm, x_ref[...], (((1,), (0,)), ((), ())),
                         preferred_element_type=jnp.float32,
                         precision=lax.Precision.HIGHEST)
    nf_ref[...] = nf
    for k in range(K_NODES):
        ns_ref[k] = jnp.max(sm[k])
    for k in range(K_NODES, 8):
        ns_ref[k] = jnp.float32(0)


_extract = pl.pallas_call(
    _extract_body,
    out_shape=(
        jax.ShapeDtypeStruct((K_NODES, D), jnp.float32),
        jax.ShapeDtypeStruct((K_NODES, N_NODES), jnp.float32),
        jax.ShapeDtypeStruct((K_NODES, D), jnp.float32),
        jax.ShapeDtypeStruct((8,), jnp.int32),
        jax.ShapeDtypeStruct((8,), jnp.float32),
    ),
    out_specs=(
        pl.BlockSpec((K_NODES, D), lambda: (0, 0)),
        pl.BlockSpec((K_NODES, N_NODES), lambda: (0, 0)),
        pl.BlockSpec((K_NODES, D), lambda: (0, 0)),
        pl.BlockSpec(memory_space=pltpu.SMEM),
        pl.BlockSpec(memory_space=pltpu.SMEM),
    ),
)


def kernel(x, edge_index, edge_weights):
    src = edge_index[0]
    dst = edge_index[1]

    w = jnp.where(edge_weights > ADJ_THRESH, edge_weights, 0.0)
    deg = jax.ops.segment_sum(w, dst, num_segments=N_NODES)
    wn = w / jnp.clip(deg[dst], 1e-12)

    perm = jnp.argsort(dst, stable=True)
    srcp = src[perm].astype(jnp.int32)
    dstp = dst[perm].astype(jnp.int32)
    wnp = wn[perm]

    pad = EPAD - E_EDGES
    srcp = jnp.concatenate([srcp, jnp.zeros((pad,), jnp.int32)])
    dstp_p = jnp.concatenate([dstp, jnp.zeros((pad,), jnp.int32)])
    wnp = jnp.concatenate([wnp, jnp.zeros((pad,), jnp.float32)])

    e0s = jnp.array([sh[0] for sh in SHARDS], jnp.int32)
    e1s = jnp.array([sh[1] for sh in SHARDS], jnp.int32)
    nxt = jnp.concatenate([
        dstp[jnp.array([sh[1] for sh in SHARDS[:-1]], jnp.int32)],
        jnp.array([N_NODES], jnp.int32)])
    meta = jnp.concatenate([e0s, e1s, nxt])

    h0 = jax.random.normal(jax.random.key(42), x.shape, dtype=jnp.float32)

    def body(i, h):
        hraw, stgrows, stgids = _sc_scatter(h, srcp, dstp_p, wnp, meta)
        hm = _merge(hraw, stgrows, stgids)
        return _normalize(hm)

    pf = lax.fori_loop(0, NUM_ITERS, body, h0)

    scores = _scores_k(pf)
    hn = _normalize(pf)
    rf, masks, nf, si8, ns8 = _extract(pf, hn, x, scores)
    return (nf, ns8[:K_NODES], masks, rf, si8[:K_NODES])
